# baseline (speedup 1.0000x reference)
import functools

import jax
import jax.numpy as jnp
from jax import lax
from jax.experimental import pallas as pl
from jax.experimental.pallas import tpu as pltpu
from jax.experimental.pallas import tpu_sc as plsc

B = 16384
D = 64
NC = 2
NS = 16
NW = NC * NS
BPW = B // NW
CHUNK = 128
NCHUNK = BPW // CHUNK


def _gather_body(ue, be, xu, xb, u_out, b_out,
                 idx_u, idx_b, urows, brows, sem_u, sem_b):
    wid = lax.axis_index("s") * NC + lax.axis_index("c")
    base = wid * BPW
    pltpu.sync_copy(xu.at[wid], idx_u)
    pltpu.sync_copy(xb.at[wid], idx_b)
    copies = []
    for j in range(NCHUNK):
        copies.append(pltpu.async_copy(
            ue.at[idx_u.at[j]], urows.at[pl.ds(j * CHUNK, CHUNK)], sem_u))
        copies.append(pltpu.async_copy(
            be.at[idx_b.at[j]], brows.at[pl.ds(j * CHUNK, CHUNK)], sem_b))
    for c in copies:
        c.wait()
    pltpu.sync_copy(urows, u_out.at[pl.ds(base, BPW)])
    pltpu.sync_copy(brows, b_out.at[pl.ds(base, BPW)])


def _sc_gather(user_emb, book_emb, xu, xb):
    mesh = plsc.VectorSubcoreMesh(
        core_axis_name="c", subcore_axis_name="s",
        num_cores=NC, num_subcores=NS)
    f = pl.kernel(
        _gather_body,
        out_type=(jax.ShapeDtypeStruct((B, D), jnp.float32),
                  jax.ShapeDtypeStruct((B, D), jnp.float32)),
        mesh=mesh,
        compiler_params=pltpu.CompilerParams(use_tc_tiling_on_sc=False),
        scratch_types=[
            pltpu.VMEM((NCHUNK, CHUNK), jnp.int32),
            pltpu.VMEM((NCHUNK, CHUNK), jnp.int32),
            pltpu.VMEM((BPW, D), jnp.float32),
            pltpu.VMEM((BPW, D), jnp.float32),
            pltpu.SemaphoreType.DMA,
            pltpu.SemaphoreType.DMA,
        ],
    )
    return f(user_emb, book_emb, xu, xb)


def _mlp_body(u_ref, b_ref, fcw_ref, fcb_ref, hlw_ref, hlb_ref, out_ref):
    h = jnp.dot(u_ref[:], fcw_ref[0:D, :], preferred_element_type=jnp.float32)
    h = h + jnp.dot(b_ref[:], fcw_ref[D:2 * D, :],
                    preferred_element_type=jnp.float32)
    h = h + fcb_ref[:]
    o = jnp.dot(h, hlw_ref[:], preferred_element_type=jnp.float32) + hlb_ref[:]
    out_ref[:] = 4.0 * jax.nn.sigmoid(o) + 1.0


def _tc_mlp(u_rows, b_rows, fc_w, fc_b, hl_w, hl_b):
    return pl.pallas_call(
        _mlp_body,
        out_shape=jax.ShapeDtypeStruct((B, 5), jnp.float32),
    )(u_rows, b_rows, fc_w, fc_b.reshape(1, -1), hl_w, hl_b.reshape(1, -1))


def kernel(x, user_emb, book_emb, fc_w, fc_b, hl_w, hl_b):
    xu = x[:, 0].reshape(NW, NCHUNK, CHUNK)
    xb = x[:, 1].reshape(NW, NCHUNK, CHUNK)
    u_rows, b_rows = _sc_gather(user_emb, book_emb, xu, xb)
    return _tc_mlp(u_rows, b_rows, fc_w, fc_b, hl_w, hl_b)


# speedup vs baseline: 2.0396x; 2.0396x over previous
import functools

import jax
import jax.numpy as jnp
from jax import lax
from jax.experimental import pallas as pl
from jax.experimental.pallas import tpu as pltpu
from jax.experimental.pallas import tpu_sc as plsc

B = 16384
D = 64
NC = 2
NS = 16
NW = NC * NS
BPW = B // NW
G = 2
NG = BPW // G
LANES = 16
SLABW = 128


def _extract_column(slabs, rows, slot, i, lane):
    lane_v = jnp.full((LANES,), lane, dtype=jnp.int32)
    slot_v = jnp.full((LANES,), slot, dtype=jnp.int32)
    iota = lax.iota(jnp.int32, LANES)
    for jj in range(D // LANES):
        k_v = iota + (jj * LANES)
        vals = plsc.load_gather(slabs, [slot_v, k_v, lane_v])
        rows[i, pl.ds(jj * LANES, LANES)] = vals


VPG = 16
NV = BPW // VPG


def _gather_table(tbl_t, x_flat, out, base, idx_vm, slabs, rows, sem_a,
                  sem_b):
    pltpu.sync_copy(x_flat.at[pl.ds(base, BPW)], idx_vm)

    def fire(r, slot, sem):
        c = lax.div(r, SLABW)
        return pltpu.async_copy(
            tbl_t.at[:, pl.ds(pl.multiple_of(c * SLABW, SLABW), SLABW)],
            slabs.at[slot], sem)

    def drain(slot, sem):
        pltpu.make_async_copy(tbl_t.at[:, pl.ds(0, SLABW)],
                              slabs.at[slot], sem).wait()

    def body(v, carry):
        iv = idx_vm[pl.ds(pl.multiple_of(v * VPG, VPG), VPG)]
        rs = [iv[b] for b in range(VPG)]
        sems = [sem_a, sem_b]
        fire(rs[0], 0, sems[0])
        fire(rs[1], 1, sems[0])
        for g in range(VPG // G):
            cur, nxt = sems[g % 2], sems[1 - g % 2]
            ch, nh = (g % 2) * G, (1 - g % 2) * G
            if (g + 1) * G < VPG:
                fire(rs[(g + 1) * G], nh, nxt)
                fire(rs[(g + 1) * G + 1], nh + 1, nxt)
            drain(ch, cur)
            drain(ch + 1, cur)
            for b in range(G):
                i = v * VPG + g * G + b
                lane = lax.rem(rs[g * G + b], SLABW)
                _extract_column(slabs, rows, ch + b, i, lane)
        return carry

    lax.fori_loop(0, NV, body, 0)
    pltpu.sync_copy(rows, out.at[pl.ds(base, BPW)])


def _gather_body(uet, bet, xu, xb, u_out, b_out, idx_vm, slabs, rows, sem_a,
                 sem_b):
    wid = lax.axis_index("s") * NC + lax.axis_index("c")
    base = wid * BPW
    _gather_table(uet, xu, u_out, base, idx_vm, slabs, rows, sem_a, sem_b)
    _gather_table(bet, xb, b_out, base, idx_vm, slabs, rows, sem_a, sem_b)


def _sc_gather(uet, bet, xu, xb):
    mesh = plsc.VectorSubcoreMesh(
        core_axis_name="c", subcore_axis_name="s",
        num_cores=NC, num_subcores=NS)
    f = pl.kernel(
        _gather_body,
        out_type=(jax.ShapeDtypeStruct((B, D), jnp.float32),
                  jax.ShapeDtypeStruct((B, D), jnp.float32)),
        mesh=mesh,
        compiler_params=pltpu.CompilerParams(needs_layout_passes=False),
        scratch_types=[
            pltpu.VMEM((BPW,), jnp.int32),
            pltpu.VMEM((2 * G, D, SLABW), jnp.float32),
            pltpu.VMEM((BPW, D), jnp.float32),
            pltpu.SemaphoreType.DMA,
            pltpu.SemaphoreType.DMA,
        ],
    )
    return f(uet, bet, xu, xb)


def _mlp_body(u_ref, b_ref, fcw_ref, fcb_ref, hlw_ref, hlb_ref, out_ref):
    h = jnp.dot(u_ref[:], fcw_ref[0:D, :], preferred_element_type=jnp.float32)
    h = h + jnp.dot(b_ref[:], fcw_ref[D:2 * D, :],
                    preferred_element_type=jnp.float32)
    h = h + fcb_ref[:]
    o = jnp.dot(h, hlw_ref[:], preferred_element_type=jnp.float32) + hlb_ref[:]
    out_ref[:] = 4.0 * jax.nn.sigmoid(o) + 1.0


def _tc_mlp(u_rows, b_rows, fc_w, fc_b, hl_w, hl_b):
    return pl.pallas_call(
        _mlp_body,
        out_shape=jax.ShapeDtypeStruct((B, 5), jnp.float32),
    )(u_rows, b_rows, fc_w, fc_b.reshape(1, -1), hl_w, hl_b.reshape(1, -1))


def kernel(x, user_emb, book_emb, fc_w, fc_b, hl_w, hl_b):
    xu = x[:, 0]
    xb = x[:, 1]
    u_rows, b_rows = _sc_gather(user_emb.T, book_emb.T, xu, xb)
    return _tc_mlp(u_rows, b_rows, fc_w, fc_b, hl_w, hl_b)


# speedup vs baseline: 2.2196x; 1.0883x over previous
import functools

import jax
import jax.numpy as jnp
from jax import lax
from jax.experimental import pallas as pl
from jax.experimental.pallas import tpu as pltpu
from jax.experimental.pallas import tpu_sc as plsc

B = 16384
D = 64
NC = 2
NS = 16
NW = NC * NS
BPW = B // NW
LANES = 16
SLABW = 128
VPG = 16
NHALF = 4
NVH = BPW // VPG // NHALF
RING = 8


def _extract_column(slabs, rows, slot, i, lane):
    lane_v = jnp.full((LANES,), lane, dtype=jnp.int32)
    slot_v = jnp.full((LANES,), slot, dtype=jnp.int32)
    iota = lax.iota(jnp.int32, LANES)
    for jj in range(D // LANES):
        k_v = iota + (jj * LANES)
        vals = plsc.load_gather(slabs, [slot_v, k_v, lane_v])
        rows[i, pl.ds(jj * LANES, LANES)] = vals


def _gather_body(uet, bet, xu, xb, u_out, b_out,
                 idxu, idxb, slabs, urows, brows, sem_a, sem_b):
    wid = lax.axis_index("s") * NC + lax.axis_index("c")
    base = wid * BPW
    pltpu.sync_copy(xu.at[pl.ds(base, BPW)], idxu)
    pltpu.sync_copy(xb.at[pl.ds(base, BPW)], idxb)

    def fire(tbl, r, slot, sem):
        c = lax.div(r, SLABW)
        return pltpu.async_copy(
            tbl.at[:, pl.ds(pl.multiple_of(c * SLABW, SLABW), SLABW)],
            slabs.at[slot], sem)

    def drain(slot, sem):
        pltpu.make_async_copy(uet.at[:, pl.ds(0, SLABW)],
                              slabs.at[slot], sem).wait()

    sems = [sem_a, sem_b]

    for half in range(NHALF):
        def body(vv, carry, half=half):
            off = (half * NVH + vv) * VPG
            ivu = idxu[pl.ds(pl.multiple_of(off, VPG), VPG)]
            ivb = idxb[pl.ds(pl.multiple_of(off, VPG), VPG)]
            rsu = [ivu[b] for b in range(VPG)]
            rsb = [ivb[b] for b in range(VPG)]

            def fire4(g, sem):
                h = (g % 2) * 4
                fire(uet, rsu[2 * g], h, sem)
                fire(uet, rsu[2 * g + 1], h + 1, sem)
                fire(bet, rsb[2 * g], h + 2, sem)
                fire(bet, rsb[2 * g + 1], h + 3, sem)

            fire4(0, sems[0])
            for g in range(VPG // 2):
                cur = sems[g % 2]
                ch = (g % 2) * 4
                if g + 1 < VPG // 2:
                    fire4(g + 1, sems[1 - g % 2])
                for s in range(4):
                    drain(ch + s, cur)
                iloc = vv * VPG + 2 * g
                _extract_column(slabs, urows, ch, iloc,
                                lax.rem(rsu[2 * g], SLABW))
                _extract_column(slabs, urows, ch + 1, iloc + 1,
                                lax.rem(rsu[2 * g + 1], SLABW))
                _extract_column(slabs, brows, ch + 2, iloc,
                                lax.rem(rsb[2 * g], SLABW))
                _extract_column(slabs, brows, ch + 3, iloc + 1,
                                lax.rem(rsb[2 * g + 1], SLABW))
            return carry

        lax.fori_loop(0, NVH, body, 0)
        hb = base + half * (BPW // NHALF)
        pltpu.sync_copy(urows, u_out.at[pl.ds(hb, BPW // NHALF)])
        pltpu.sync_copy(brows, b_out.at[pl.ds(hb, BPW // NHALF)])


def _sc_gather(uet, bet, xu, xb):
    mesh = plsc.VectorSubcoreMesh(
        core_axis_name="c", subcore_axis_name="s",
        num_cores=NC, num_subcores=NS)
    f = pl.kernel(
        _gather_body,
        out_type=(jax.ShapeDtypeStruct((B, D), jnp.float32),
                  jax.ShapeDtypeStruct((B, D), jnp.float32)),
        mesh=mesh,
        compiler_params=pltpu.CompilerParams(needs_layout_passes=False),
        scratch_types=[
            pltpu.VMEM((BPW,), jnp.int32),
            pltpu.VMEM((BPW,), jnp.int32),
            pltpu.VMEM((RING, D, SLABW), jnp.float32),
            pltpu.VMEM((BPW // NHALF, D), jnp.float32),
            pltpu.VMEM((BPW // NHALF, D), jnp.float32),
            pltpu.SemaphoreType.DMA,
            pltpu.SemaphoreType.DMA,
        ],
    )
    return f(uet, bet, xu, xb)


def _mlp_body(u_ref, b_ref, fcw_ref, fcb_ref, hlw_ref, hlb_ref, out_ref):
    h = jnp.dot(u_ref[:], fcw_ref[0:D, :], preferred_element_type=jnp.float32)
    h = h + jnp.dot(b_ref[:], fcw_ref[D:2 * D, :],
                    preferred_element_type=jnp.float32)
    h = h + fcb_ref[:]
    o = jnp.dot(h, hlw_ref[:], preferred_element_type=jnp.float32) + hlb_ref[:]
    out_ref[:] = 4.0 * jax.nn.sigmoid(o) + 1.0


def _tc_mlp(u_rows, b_rows, fc_w, fc_b, hl_w, hl_b):
    return pl.pallas_call(
        _mlp_body,
        out_shape=jax.ShapeDtypeStruct((B, 5), jnp.float32),
    )(u_rows, b_rows, fc_w, fc_b.reshape(1, -1), hl_w, hl_b.reshape(1, -1))


def kernel(x, user_emb, book_emb, fc_w, fc_b, hl_w, hl_b):
    xu = x[:, 0]
    xb = x[:, 1]
    u_rows, b_rows = _sc_gather(user_emb.T, book_emb.T, xu, xb)
    return _tc_mlp(u_rows, b_rows, fc_w, fc_b, hl_w, hl_b)
